# baseline (speedup 1.0000x reference)
import functools

import jax
import jax.numpy as jnp
from jax import lax
from jax.experimental import pallas as pl
from jax.experimental.pallas import tpu as pltpu
from jax.experimental.pallas import tpu_sc as plsc

_NC = 2
_NS = 16
_R = 2
_SPB = 64
_SCS_ROWS = 512


def kernel(x, pe):
    batch, seq_len = x.shape
    max_len, d_model = pe.shape
    flat = seq_len * d_model
    pe_flat = pe.reshape(1, flat)
    rows_per_core = batch // _NC
    tec_rows = rows_per_core - _SCS_ROWS
    rows_per_w = tec_rows // _NS
    n_chunks = rows_per_w // _R
    n_scs_chunks = _SCS_ROWS // _SPB

    vec_mesh = plsc.VectorSubcoreMesh(
        core_axis_name="c", subcore_axis_name="s"
    )
    scs_mesh = plsc.ScalarSubcoreMesh(axis_name="c", num_cores=_NC)

    def tec_body(pe_hbm, out_hbm, buf, shared, vsem, ssem):
        del shared, ssem
        cid = lax.axis_index("c")
        sid = lax.axis_index("s")
        base = cid * rows_per_core + sid * rows_per_w
        for r in range(_R):
            pltpu.sync_copy(pe_hbm.at[0], buf.at[r])
        for j in range(n_chunks):
            pltpu.make_async_copy(
                buf, out_hbm.at[pl.ds(base + j * _R, _R)], vsem
            ).start()
        for j in range(n_chunks):
            pltpu.make_async_copy(
                buf, out_hbm.at[pl.ds(base + j * _R, _R)], vsem
            ).wait()

    def scs_body(pe_hbm, out_hbm, buf, shared, vsem, ssem):
        del buf, vsem
        cid = lax.axis_index("c")
        base = cid * rows_per_core + tec_rows
        for r in range(_SPB):
            pltpu.sync_copy(pe_hbm.at[0], shared.at[r])
        for j in range(n_scs_chunks):
            pltpu.make_async_copy(
                shared, out_hbm.at[pl.ds(base + j * _SPB, _SPB)], ssem
            ).start()
        for j in range(n_scs_chunks):
            pltpu.make_async_copy(
                shared, out_hbm.at[pl.ds(base + j * _SPB, _SPB)], ssem
            ).wait()

    run = pl.kernel(
        [tec_body, scs_body],
        out_type=jax.ShapeDtypeStruct((batch, flat), jnp.float32),
        mesh=[vec_mesh, scs_mesh],
        scratch_types=[
            (pltpu.MemorySpace.VMEM @ vec_mesh)((_R, flat), jnp.float32),
            pltpu.MemorySpace.VMEM_SHARED((_SPB, flat), jnp.float32),
            pltpu.SemaphoreType.DMA @ vec_mesh,
            pltpu.SemaphoreType.DMA @ scs_mesh,
        ],
    )

    out = run(pe_flat)
    return out.reshape(batch, seq_len, d_model)


# speedup vs baseline: 1.1462x; 1.1462x over previous
import functools

import jax
import jax.numpy as jnp
from jax import lax
from jax.experimental import pallas as pl
from jax.experimental.pallas import tpu as pltpu
from jax.experimental.pallas import tpu_sc as plsc

_NC = 2
_NS = 16


def kernel(x, pe):
    batch, seq_len = x.shape
    max_len, d_model = pe.shape
    flat = seq_len * d_model
    pe_flat = pe.reshape(1, flat)
    nw = _NC * _NS
    rows_per_w = batch // nw

    mesh = plsc.VectorSubcoreMesh(core_axis_name="c", subcore_axis_name="s")

    @functools.partial(
        pl.kernel,
        mesh=mesh,
        out_type=jax.ShapeDtypeStruct((batch, flat), jnp.float32),
        scratch_types=[
            pltpu.VMEM((1, flat), jnp.float32),
            pltpu.SemaphoreType.DMA,
        ],
    )
    def sc_bcast(pe_hbm, out_hbm, buf, sem):
        wid = lax.axis_index("s") * _NC + lax.axis_index("c")
        pltpu.sync_copy(pe_hbm.at[0], buf.at[0])
        for j in range(rows_per_w):
            pltpu.make_async_copy(
                buf, out_hbm.at[pl.ds(j * nw + wid, 1)], sem
            ).start()
        for j in range(rows_per_w):
            pltpu.make_async_copy(
                buf, out_hbm.at[pl.ds(j * nw + wid, 1)], sem
            ).wait()

    out = sc_bcast(pe_flat)
    return out.reshape(batch, seq_len, d_model)


# speedup vs baseline: 1.1474x; 1.0010x over previous
import functools

import jax
import jax.numpy as jnp
from jax import lax
from jax.experimental import pallas as pl
from jax.experimental.pallas import tpu as pltpu
from jax.experimental.pallas import tpu_sc as plsc

_NC = 2
_NS = 16


def kernel(x, pe):
    batch, seq_len = x.shape
    max_len, d_model = pe.shape
    flat = seq_len * d_model
    pe_flat = pe.reshape(1, flat)
    nw = _NC * _NS
    rows_per_w = batch // nw

    mesh = plsc.VectorSubcoreMesh(core_axis_name="c", subcore_axis_name="s")

    @functools.partial(
        pl.kernel,
        mesh=mesh,
        out_type=jax.ShapeDtypeStruct((batch, flat), jnp.float32),
        scratch_types=[
            pltpu.VMEM((1, flat), jnp.float32),
            pltpu.SemaphoreType.DMA,
        ],
    )
    def sc_bcast(pe_hbm, out_hbm, buf, sem):
        wid = lax.axis_index("s") * _NC + lax.axis_index("c")
        base = wid * rows_per_w
        pltpu.sync_copy(pe_hbm.at[0], buf.at[0])
        for j in range(rows_per_w):
            pltpu.make_async_copy(
                buf, out_hbm.at[pl.ds(base + j, 1)], sem
            ).start()
        for j in range(rows_per_w):
            pltpu.make_async_copy(
                buf, out_hbm.at[pl.ds(base + j, 1)], sem
            ).wait()

    out = sc_bcast(pe_flat)
    return out.reshape(batch, seq_len, d_model)
